# Initial kernel scaffold; baseline (speedup 1.0000x reference)
import jax
import jax.numpy as jnp
from jax.experimental import pallas as pl


def kernel(x, edge_index, W1, W2, Wo, bo):
    raise NotImplementedError("write your pallas kernel here")


# speedup vs baseline: 4.9839x; 4.9839x over previous
import jax
import jax.numpy as jnp
from jax import lax
from jax.experimental import pallas as pl
from jax.experimental.pallas import tpu as pltpu
from jax.experimental.pallas import tpu_sc as plsc

N = 10000
NP = 10240
E = 320000
D = 128
NC = 2
NS = 16
NW = NC * NS
EPW = E // NW
K = 80
NCH = EPW // K
RPS = NP // NS


def _make_sc_agg(with_deg: bool):
    out_type = [jax.ShapeDtypeStruct((NC, NP, D), jnp.float32)]
    scratch = [
        pltpu.VMEM_SHARED((NP, D), jnp.float32),
        pltpu.VMEM((2, K), jnp.int32),
        pltpu.VMEM((K, D), jnp.float32),
        pltpu.SemaphoreType.DMA,
    ]
    if with_deg:
        out_type.append(jax.ShapeDtypeStruct((NC * NP,), jnp.float32))
        scratch += [
            pltpu.VMEM_SHARED((NP,), jnp.float32),
            pltpu.VMEM((K,), jnp.float32),
        ]

    mesh = plsc.VectorSubcoreMesh(core_axis_name="c", subcore_axis_name="s")

    def body(*refs):
        if with_deg:
            (x, srcr, dstr, zr, z1r, onesr,
             part_o, deg_o, acc, eiv, rows, sem, dacc, onesv) = refs
        else:
            (x, srcr, dstr, zr,
             part_o, acc, eiv, rows, sem) = refs
        c = lax.axis_index("c")
        s = lax.axis_index("s")
        r0 = s * RPS
        pltpu.sync_copy(zr, acc.at[pl.ds(r0, RPS)])
        if with_deg:
            pltpu.sync_copy(z1r, dacc.at[pl.ds(r0, RPS)])
            pltpu.sync_copy(onesr, onesv)
        plsc.subcore_barrier()
        base = (c * NS + s) * EPW

        def step(i, carry):
            off = base + i * K
            pltpu.sync_copy(srcr.at[pl.ds(off, K)], eiv.at[0])
            pltpu.sync_copy(dstr.at[pl.ds(off, K)], eiv.at[1])
            pltpu.async_copy(x.at[eiv.at[0]], rows, sem).wait()
            pltpu.sync_copy(rows, acc.at[eiv.at[1]], add=True)
            if with_deg:
                pltpu.sync_copy(onesv, dacc.at[eiv.at[1]], add=True)
            return carry

        lax.fori_loop(0, NCH, step, 0)
        plsc.subcore_barrier()
        pltpu.sync_copy(acc.at[pl.ds(r0, RPS)], part_o.at[c, pl.ds(r0, RPS)])
        if with_deg:
            pltpu.sync_copy(dacc.at[pl.ds(r0, RPS)],
                            deg_o.at[pl.ds(c * NP + r0, RPS)])

    return pl.kernel(body, out_type=tuple(out_type), mesh=mesh,
                     scratch_types=scratch)


_sc_agg_deg = _make_sc_agg(True)
_sc_agg = _make_sc_agg(False)

BR = 1024


def _tc1_body(p_ref, d_ref, w_ref, o_ref):
    ssum = p_ref[0] + p_ref[1]
    agg = ssum / jnp.maximum(d_ref[...], 1.0)
    h = lax.dot_general(agg, w_ref[...], (((1,), (1,)), ((), ())),
                        preferred_element_type=jnp.float32)
    o_ref[...] = jnp.maximum(h, 0.0)


def _tc2_body(p_ref, d_ref, w2_ref, wo_ref, bo_ref, out_ref, h_ref):
    ssum = p_ref[0] + p_ref[1]
    agg = ssum / jnp.maximum(d_ref[...], 1.0)
    h2 = lax.dot_general(agg, w2_ref[...], (((1,), (1,)), ((), ())),
                         preferred_element_type=jnp.float32)
    logits = lax.dot_general(h2, wo_ref[...], (((1,), (1,)), ((), ())),
                             preferred_element_type=jnp.float32)
    logits = logits + bo_ref[...]
    m = jnp.max(logits, axis=1, keepdims=True)
    lse = jnp.log(jnp.sum(jnp.exp(logits - m), axis=1, keepdims=True)) + m
    out_ref[...] = logits - lse
    h_ref[...] = h2


def _tc_layer1(part, degcol, W1):
    return pl.pallas_call(
        _tc1_body,
        grid=(NP // BR,),
        in_specs=[
            pl.BlockSpec((NC, BR, D), lambda i: (0, i, 0)),
            pl.BlockSpec((BR, 1), lambda i: (i, 0)),
            pl.BlockSpec((D, D), lambda i: (0, 0)),
        ],
        out_specs=pl.BlockSpec((BR, D), lambda i: (i, 0)),
        out_shape=jax.ShapeDtypeStruct((NP, D), jnp.float32),
    )(part, degcol, W1)


def _tc_layer2(part, degcol, W2, Wo, bo2d):
    return pl.pallas_call(
        _tc2_body,
        grid=(NP // BR,),
        in_specs=[
            pl.BlockSpec((NC, BR, D), lambda i: (0, i, 0)),
            pl.BlockSpec((BR, 1), lambda i: (i, 0)),
            pl.BlockSpec((D, D), lambda i: (0, 0)),
            pl.BlockSpec((64, D), lambda i: (0, 0)),
            pl.BlockSpec((1, 64), lambda i: (0, 0)),
        ],
        out_specs=[
            pl.BlockSpec((BR, 64), lambda i: (i, 0)),
            pl.BlockSpec((BR, D), lambda i: (i, 0)),
        ],
        out_shape=[
            jax.ShapeDtypeStruct((NP, 64), jnp.float32),
            jax.ShapeDtypeStruct((NP, D), jnp.float32),
        ],
    )(part, degcol, W2, Wo, bo2d)


def kernel(x, edge_index, W1, W2, Wo, bo):
    src = edge_index[0]
    dst = edge_index[1]
    zeros = jnp.zeros((RPS, D), jnp.float32)
    z1 = jnp.zeros((RPS,), jnp.float32)
    ones = jnp.ones((K,), jnp.float32)
    part1, deg1 = _sc_agg_deg(x, src, dst, zeros, z1, ones)
    degcol = (deg1[:NP] + deg1[NP:]).reshape(NP, 1)
    h1 = _tc_layer1(part1, degcol, W1)
    (part2,) = _sc_agg(h1, src, dst, zeros)
    out, h2 = _tc_layer2(part2, degcol, W2, Wo, bo.reshape(1, 64))
    return (out[:N], h2[:N])
